# Initial kernel scaffold; baseline (speedup 1.0000x reference)
import jax
import jax.numpy as jnp
from jax.experimental import pallas as pl


def kernel(x, edge_index, W, att_src, att_dst, bias):
    raise NotImplementedError("write your pallas kernel here")


# speedup vs baseline: 13.7512x; 13.7512x over previous
import functools

import jax
import jax.numpy as jnp
from jax.experimental import pallas as pl


def _gat_kernel(x_ref, ei_ref, w_ref, acat_ref, bias_ref, out_ref, *, G, J, H, C, Eper):
    D = H * C
    src = ei_ref[0:1, :]
    dst = ei_ref[1:2, :]
    node_iota = jax.lax.broadcasted_iota(jnp.int32, (J, Eper), 0)
    src_oh = (node_iota == src).astype(jnp.float32)
    dst_oh = (node_iota == dst).astype(jnp.float32)
    counts = jax.lax.dot_general(
        dst_oh, src_oh, (((1,), (1,)), ((), ())),
        preferred_element_type=jnp.float32)
    r_iota = jax.lax.broadcasted_iota(jnp.int32, (J, J), 0)
    c_iota = jax.lax.broadcasted_iota(jnp.int32, (J, J), 1)
    A = counts + (r_iota == c_iota).astype(jnp.float32)

    xb = x_ref[...]
    x2 = xb.reshape(G * J, D)
    h = jnp.dot(x2, w_ref[...], preferred_element_type=jnp.float32)
    a = jnp.dot(h, acat_ref[...], preferred_element_type=jnp.float32)
    a3 = a.reshape(G, J, 2 * H)
    a_src = a3[:, :, :H]
    a_dst = a3[:, :, H:]

    alpha = a_src[:, None, :, :] + a_dst[:, :, None, :]
    alpha = jnp.where(alpha >= 0, alpha, 0.2 * alpha)
    Ab = A[None, :, :, None]
    masked = jnp.where(Ab > 0, alpha, -1e30)
    amax = jnp.max(masked, axis=2, keepdims=True)
    ex = jnp.exp(masked - amax) * Ab
    denom = jnp.sum(ex, axis=2, keepdims=True)
    P = ex / (denom + 1e-16)

    hf = h.reshape(G, J, H, C)
    outs = []
    for hh in range(H):
        Ph = P[:, :, :, hh]
        Hh = hf[:, :, hh, :]
        oh = jax.lax.dot_general(
            Ph, Hh, (((2,), (1,)), ((0,), (0,))),
            preferred_element_type=jnp.float32)
        outs.append(oh)
    out = jnp.concatenate(outs, axis=-1)
    out_ref[...] = out + bias_ref[...][None, :, :]


@jax.jit
def kernel(x, edge_index, W, att_src, att_dst, bias):
    b, t, j, d = x.shape
    BT = b * t
    H = att_src.shape[1]
    C = att_src.shape[2]
    Eper = edge_index.shape[1]
    G = 16

    x3 = x.reshape(BT, j, d)
    eyeH = jnp.eye(H, dtype=jnp.float32)
    asrc_mat = (att_src.reshape(H, C)[:, :, None] * eyeH[:, None, :]).reshape(d, H)
    adst_mat = (att_dst.reshape(H, C)[:, :, None] * eyeH[:, None, :]).reshape(d, H)
    acat = jnp.concatenate([asrc_mat, adst_mat], axis=1)
    bias2 = bias.reshape(1, d)

    grid = (BT // G,)
    out = pl.pallas_call(
        functools.partial(_gat_kernel, G=G, J=j, H=H, C=C, Eper=Eper),
        grid=grid,
        in_specs=[
            pl.BlockSpec((G, j, d), lambda i: (i, 0, 0)),
            pl.BlockSpec((2, Eper), lambda i: (0, 0)),
            pl.BlockSpec((d, d), lambda i: (0, 0)),
            pl.BlockSpec((d, 2 * H), lambda i: (0, 0)),
            pl.BlockSpec((1, d), lambda i: (0, 0)),
        ],
        out_specs=pl.BlockSpec((G, j, d), lambda i: (i, 0, 0)),
        out_shape=jax.ShapeDtypeStruct((BT, j, d), jnp.float32),
    )(x3, edge_index, W, acat, bias2)
    return out.reshape(b, t, j, d)


# speedup vs baseline: 38.4857x; 2.7987x over previous
import functools

import jax
import jax.numpy as jnp
from jax.experimental import pallas as pl


def _gat_kernel(x_ref, ei_ref, w_ref, acat_ref, bias_ref, out_ref, *, G, J, H, C, Eper):
    D = H * C
    R = G * J
    f32 = jnp.float32

    row_node = jax.lax.broadcasted_iota(jnp.int32, (R, Eper), 0) % J
    src = ei_ref[0:1, :]
    dst = ei_ref[1:2, :]
    src_oh = (row_node == src).astype(f32)
    dst_oh = (row_node == dst).astype(f32)
    tiledA = jax.lax.dot_general(
        dst_oh, src_oh, (((1,), (1,)), ((), ())),
        preferred_element_type=f32)
    ri = jax.lax.broadcasted_iota(jnp.int32, (R, R), 0)
    ci = jax.lax.broadcasted_iota(jnp.int32, (R, R), 1)
    same_graph = (ri // J) == (ci // J)
    Abig = jnp.where(same_graph, tiledA, 0.0) + (ri == ci).astype(f32)
    valid = Abig > 0

    x2 = x_ref[...].reshape(R, D)
    h = jnp.dot(x2, w_ref[...], preferred_element_type=f32)
    a = jnp.dot(h, acat_ref[...], preferred_element_type=f32)

    ones = jnp.ones((R, 1), dtype=f32)
    outs = []
    for hh in range(H):
        a_src_h = a[:, hh:hh + 1]
        a_dst_h = a[:, H + hh:H + hh + 1]
        lhs = jnp.concatenate([a_dst_h, ones], axis=1)
        rhs = jnp.concatenate([ones, a_src_h], axis=1)
        alpha = jax.lax.dot_general(
            lhs, rhs, (((1,), (1,)), ((), ())),
            preferred_element_type=f32)
        alpha = jnp.where(alpha >= 0, alpha, 0.2 * alpha)
        masked = jnp.where(valid, alpha, -1e30)
        amax = jnp.max(masked, axis=1, keepdims=True)
        ex = jnp.exp(masked - amax) * Abig
        denom = jnp.sum(ex, axis=1, keepdims=True)
        P = ex / (denom + 1e-16)
        out_h = jnp.dot(P, h[:, hh * C:(hh + 1) * C],
                        preferred_element_type=f32)
        outs.append(out_h)
    out = jnp.concatenate(outs, axis=-1)
    out = out + bias_ref[...]
    out_ref[...] = out.reshape(G, J, D)


@jax.jit
def kernel(x, edge_index, W, att_src, att_dst, bias):
    b, t, j, d = x.shape
    BT = b * t
    H = att_src.shape[1]
    C = att_src.shape[2]
    Eper = edge_index.shape[1]
    G = 5

    x3 = x.reshape(BT, j, d)
    eyeH = jnp.eye(H, dtype=jnp.float32)
    asrc_mat = (att_src.reshape(H, C)[:, :, None] * eyeH[:, None, :]).reshape(d, H)
    adst_mat = (att_dst.reshape(H, C)[:, :, None] * eyeH[:, None, :]).reshape(d, H)
    acat = jnp.concatenate([asrc_mat, adst_mat], axis=1)
    bias2 = bias.reshape(1, d)

    grid = (BT // G,)
    out = pl.pallas_call(
        functools.partial(_gat_kernel, G=G, J=j, H=H, C=C, Eper=Eper),
        grid=grid,
        in_specs=[
            pl.BlockSpec((G, j, d), lambda i: (i, 0, 0)),
            pl.BlockSpec((2, Eper), lambda i: (0, 0)),
            pl.BlockSpec((d, d), lambda i: (0, 0)),
            pl.BlockSpec((d, 2 * H), lambda i: (0, 0)),
            pl.BlockSpec((1, d), lambda i: (0, 0)),
        ],
        out_specs=pl.BlockSpec((G, j, d), lambda i: (i, 0, 0)),
        out_shape=jax.ShapeDtypeStruct((BT, j, d), jnp.float32),
    )(x3, edge_index, W, acat, bias2)
    return out.reshape(b, t, j, d)
